# Initial kernel scaffold; baseline (speedup 1.0000x reference)
import jax
import jax.numpy as jnp
from jax.experimental import pallas as pl


def kernel(word_pos, ctx_pos, neg_ctx_pos, word_table, ctx_table):
    raise NotImplementedError("write your pallas kernel here")


# speedup vs baseline: 7.0977x; 7.0977x over previous
import functools

import jax
import jax.numpy as jnp
from jax import lax
from jax.experimental import pallas as pl
from jax.experimental.pallas import tpu as pltpu
from jax.experimental.pallas import tpu_sc as plsc

VOCAB = 100000
EMBED = 128
B = 16384
NNEG = 20
NCTX = NNEG + 1
NLANE = 16
NREG = EMBED // NLANE
SROW = 32

NC = 2
NS = 16
NW = NC * NS
RW = B // NW
C = 16
NCHUNK = RW // C
CI = C * NCTX

_DNUMS = lax.GatherDimensionNumbers(
    offset_dims=(), collapsed_slice_dims=(0,), start_index_map=(0,))


def _lane_rot(p, sh):
  perm = ((lax.iota(jnp.int32, NLANE) + sh) % NLANE)[:, None]
  return lax.gather(p, perm, _DNUMS, (1,),
                    mode=lax.GatherScatterMode.PROMISE_IN_BOUNDS)


def _allsum(p):
  for sh in (8, 4, 2, 1):
    p = p + _lane_rot(p, sh)
  return p


def _sc_scores(word_pos, cidx, word_table, ctx_table):
  mesh = plsc.VectorSubcoreMesh(core_axis_name="c", subcore_axis_name="s")

  @functools.partial(
      pl.kernel,
      mesh=mesh,
      out_type=jax.ShapeDtypeStruct((B * SROW,), jnp.float32),
      scratch_types=[
          pltpu.VMEM((RW,), jnp.int32),
          pltpu.VMEM((RW * NCTX,), jnp.int32),
          pltpu.VMEM((C, EMBED), jnp.float32),
          pltpu.VMEM((CI, EMBED), jnp.float32),
          pltpu.VMEM((RW * SROW,), jnp.float32),
          pltpu.SemaphoreType.DMA,
      ],
  )
  def k(wp_hbm, cidx_hbm, wt_hbm, ct_hbm, out_hbm,
        widx_v, cidx_v, wrows_v, crows_v, sbuf_v, sem):
    wid = lax.axis_index("s") * NC + lax.axis_index("c")
    base = pl.multiple_of(wid * RW, RW)
    pltpu.sync_copy(wp_hbm.at[pl.ds(base, RW)], widx_v)
    pltpu.sync_copy(cidx_hbm.at[pl.ds(base * NCTX, RW * NCTX)], cidx_v)

    lane = lax.iota(jnp.int32, NLANE)

    def chunk_body(c, _):
      cb = pl.multiple_of(c * C, C)
      cib = pl.multiple_of(c * CI, CI)
      cp0 = pltpu.async_copy(
          wt_hbm.at[widx_v.at[pl.ds(cb, C)]], wrows_v, sem)
      cp1 = pltpu.async_copy(
          ct_hbm.at[cidx_v.at[pl.ds(cib, 128)]],
          crows_v.at[pl.ds(0, 128)], sem)
      cp2 = pltpu.async_copy(
          ct_hbm.at[cidx_v.at[pl.ds(cib + 128, 128)]],
          crows_v.at[pl.ds(128, 128)], sem)
      cp3 = pltpu.async_copy(
          ct_hbm.at[cidx_v.at[pl.ds(cib + 256, CI - 256)]],
          crows_v.at[pl.ds(256, CI - 256)], sem)
      cp0.wait()
      cp1.wait()
      cp2.wait()
      cp3.wait()

      def row_body(i, _):
        w = [wrows_v[i, pl.ds(r * NLANE, NLANE)] for r in range(NREG)]
        s_lo = jnp.zeros((NLANE,), jnp.float32)
        s_hi = jnp.zeros((NLANE,), jnp.float32)
        for j in range(NCTX):
          crow = crows_v.at[i * NCTX + j]
          p = w[0] * crow[pl.ds(0, NLANE)]
          for r in range(1, NREG):
            p = p + w[r] * crow[pl.ds(r * NLANE, NLANE)]
          tot = _allsum(p)
          if j < NLANE:
            s_lo = jnp.where(lane == j, tot, s_lo)
          else:
            s_hi = jnp.where(lane == (j - NLANE), tot, s_hi)
        sb = (cb + i) * SROW
        sbuf_v[pl.ds(sb, NLANE)] = s_lo
        sbuf_v[pl.ds(sb + NLANE, NLANE)] = s_hi
        return 0

      lax.fori_loop(0, C, row_body, 0)
      return 0

    lax.fori_loop(0, NCHUNK, chunk_body, 0)
    pltpu.sync_copy(sbuf_v, out_hbm.at[pl.ds(base * SROW, RW * SROW)])

  return k(word_pos, cidx, word_table, ctx_table)


def _tc_finish(scores):
  def body(s_ref, o_ref):
    s = s_ref[...]
    pos = s[:, 0:1]
    neg = -s[:, 1:NCTX]

    def logsig(x):
      return jnp.minimum(x, 0.0) - jnp.log1p(jnp.exp(-jnp.abs(x)))

    o_ref[...] = -(logsig(pos)[:, 0] + jnp.sum(logsig(neg), axis=1))

  return pl.pallas_call(
      body,
      out_shape=jax.ShapeDtypeStruct((B,), jnp.float32),
  )(scores)


def kernel(word_pos, ctx_pos, neg_ctx_pos, word_table, ctx_table):
  word_pos = word_pos.astype(jnp.int32)
  cidx = jnp.concatenate(
      [ctx_pos.astype(jnp.int32)[:, None], neg_ctx_pos.astype(jnp.int32)],
      axis=1).reshape(-1)
  scores = _sc_scores(word_pos, cidx, word_table, ctx_table)
  return _tc_finish(scores.reshape(B, SROW))
